# Initial kernel scaffold; baseline (speedup 1.0000x reference)
import jax
import jax.numpy as jnp
from jax.experimental import pallas as pl


def kernel(pred, target):
    raise NotImplementedError("write your pallas kernel here")


# speedup vs baseline: 27.8962x; 27.8962x over previous
import functools

import jax
import jax.numpy as jnp
from jax import lax
from jax.experimental import pallas as pl
from jax.experimental.pallas import tpu as pltpu
from jax.experimental.pallas import tpu_sc as plsc

N = 8 * 512 * 512
NW = 32
PER_W = N // NW
CHUNK = 16384
NCHUNK = PER_W // CHUNK
KB = 2048
L = 16
HIST = L * KB
EPS = 1e-10

@functools.cache
def _build_sc_hist():
    mesh = plsc.VectorSubcoreMesh(core_axis_name="c", subcore_axis_name="s")
    return functools.partial(
        pl.kernel,
        mesh=mesh,
        out_type=[
            jax.ShapeDtypeStruct((NW, HIST), jnp.float32),
            jax.ShapeDtypeStruct((NW, HIST), jnp.float32),
            jax.ShapeDtypeStruct((NW, L), jnp.float32),
        ],
        scratch_types=[
            pltpu.VMEM((CHUNK,), jnp.float32),
            pltpu.VMEM((CHUNK,), jnp.int32),
            pltpu.VMEM((HIST,), jnp.float32),
            pltpu.VMEM((HIST,), jnp.float32),
            pltpu.VMEM((L,), jnp.float32),
        ],
        compiler_params=pltpu.CompilerParams(needs_layout_passes=False),
    )(_sc_hist_body)


def _sc_hist_body(pred_hbm, tgt_hbm, out_cnt, out_sum, out_acc,
                  pbuf, tbuf, hcnt, hsum, accbuf):
    wid = lax.axis_index("c") * 16 + lax.axis_index("s")
    zeros = jnp.zeros((L,), jnp.float32)
    ones = jnp.ones((L,), jnp.float32)
    lane = lax.iota(jnp.int32, L)

    def _zero(i, carry):
        hcnt[pl.ds(i * L, L)] = zeros
        hsum[pl.ds(i * L, L)] = zeros
        return carry

    lax.fori_loop(0, HIST // L, _zero, 0)

    def _step(i, acc):
        vp = pbuf[pl.ds(i * L, L)]
        vt = tbuf[pl.ds(i * L, L)]
        pc = jnp.minimum(jnp.maximum(vp, 0.0), 1.0)
        neg = vt == 0
        b = jnp.minimum((pc * float(KB)).astype(jnp.int32), KB - 1)
        idx = lane * KB + ((KB - 1) - b)
        plsc.addupdate_scatter(hcnt, [idx], ones, mask=neg)
        plsc.addupdate_scatter(hsum, [idx], 1.0 + pc, mask=neg)
        return acc + pc

    acc = zeros
    for ci in range(NCHUNK):
        base = wid * PER_W + ci * CHUNK
        pltpu.sync_copy(pred_hbm.at[pl.ds(base, CHUNK)], pbuf)
        pltpu.sync_copy(tgt_hbm.at[pl.ds(base, CHUNK)], tbuf)
        acc = lax.fori_loop(0, CHUNK // L, _step, acc)

    accbuf[...] = acc
    pltpu.sync_copy(hcnt, out_cnt.at[wid])
    pltpu.sync_copy(hsum, out_sum.at[wid])
    pltpu.sync_copy(accbuf, out_acc.at[wid])


def _combine_body(cnt_ref, sum_ref, acc_ref, out_ref):
    nf = float(N)
    kbf = float(KB)
    cnt = jnp.sum(cnt_ref[...], axis=0)
    ssum = jnp.sum(sum_ref[...], axis=0)
    acc_pc = jnp.sum(acc_ref[...])
    n_neg = jnp.sum(cnt)
    s_neg = jnp.sum(ssum)
    gts = nf - n_neg
    s_pos = (nf - acc_pc) - (2.0 * n_neg - s_neg)
    term1 = s_pos / (nf + EPS)

    iu0 = lax.broadcasted_iota(jnp.int32, (128, 128), 0)
    iu1 = lax.broadcasted_iota(jnp.int32, (128, 128), 1)
    upper = (iu0 <= iu1).astype(jnp.float32)
    im0 = lax.broadcasted_iota(jnp.int32, (16, 16), 0)
    im1 = lax.broadcasted_iota(jnp.int32, (16, 16), 1)
    strict_lower = (im0 > im1).astype(jnp.float32)
    ones128 = jnp.ones((128, 128), jnp.float32)
    incl = jnp.dot(cnt, upper, preferred_element_type=jnp.float32)
    rowtot_b = jnp.dot(cnt, ones128, preferred_element_type=jnp.float32)
    excl_rows = jnp.dot(strict_lower, rowtot_b,
                        preferred_element_type=jnp.float32)
    j_excl = excl_rows + incl - cnt

    a = gts + j_excl + EPS
    term2 = jnp.sum(gts * ssum / (a * (a + cnt)))

    bidx = (lax.broadcasted_iota(jnp.int32, (16, 128), 0) * 128
            + lax.broadcasted_iota(jnp.int32, (16, 128), 1)).astype(jnp.float32)
    emax = jnp.max(jnp.where(cnt > 0.0, 1.0 + (kbf - bidx) / kbf, -1.0))
    loss = term1 + term2 + jnp.where(gts == 0.0, emax, 0.0)
    out_ref[0, 0] = loss


_combine = pl.pallas_call(
    _combine_body,
    out_shape=jax.ShapeDtypeStruct((1, 1), jnp.float32),
    out_specs=pl.BlockSpec(memory_space=pltpu.SMEM),
)


def kernel(pred, target):
    predf = pred.reshape(N)
    tgt = target.reshape(N)
    cnt, ssum, acc = _build_sc_hist()(predf, tgt)
    cnt3 = cnt.reshape(NW * L, KB // 128, 128)
    sum3 = ssum.reshape(NW * L, KB // 128, 128)
    loss = _combine(cnt3, sum3, acc)
    return loss[0, 0]


# speedup vs baseline: 43.2877x; 1.5517x over previous
import functools

import jax
import jax.numpy as jnp
from jax import lax
from jax.experimental import pallas as pl
from jax.experimental.pallas import tpu as pltpu
from jax.experimental.pallas import tpu_sc as plsc

N = 8 * 512 * 512
NW = 32
PER_W = N // NW
CHUNK = 16384
NCHUNK = PER_W // CHUNK
KB = 2048
L = 16
HIST = L * KB
EPS = 1e-10

@functools.cache
def _build_sc_hist():
    mesh = plsc.VectorSubcoreMesh(core_axis_name="c", subcore_axis_name="s")
    return functools.partial(
        pl.kernel,
        mesh=mesh,
        out_type=[
            jax.ShapeDtypeStruct((NW, HIST), jnp.float32),
            jax.ShapeDtypeStruct((NW, HIST), jnp.float32),
            jax.ShapeDtypeStruct((NW, L), jnp.float32),
        ],
        scratch_types=[
            pltpu.VMEM((CHUNK,), jnp.float32),
            pltpu.VMEM((CHUNK,), jnp.int32),
            pltpu.VMEM((HIST,), jnp.float32),
            pltpu.VMEM((HIST,), jnp.float32),
            pltpu.VMEM((L,), jnp.float32),
        ],
        compiler_params=pltpu.CompilerParams(needs_layout_passes=False),
    )(_sc_hist_body)


def _sc_hist_body(pred_hbm, tgt_hbm, out_cnt, out_sum, out_acc,
                  pbuf, tbuf, hcnt, hsum, accbuf):
    wid = lax.axis_index("c") * 16 + lax.axis_index("s")
    zeros = jnp.zeros((L,), jnp.float32)
    ones = jnp.ones((L,), jnp.float32)
    lane = lax.iota(jnp.int32, L)
    idxbase = lane * KB + (KB - 1)

    accbuf[...] = zeros

    @plsc.parallel_loop(0, HIST // L, unroll=8)
    def _zero(i):
        hcnt[pl.ds(i * L, L)] = zeros
        hsum[pl.ds(i * L, L)] = zeros

    def _step(i):
        vp = pbuf[pl.ds(i * L, L)]
        vt = tbuf[pl.ds(i * L, L)]
        pc = jnp.minimum(jnp.maximum(vp, 0.0), 1.0)
        neg = vt == 0
        b = jnp.minimum((pc * float(KB)).astype(jnp.int32), KB - 1)
        idx = idxbase - b
        plsc.addupdate_scatter(hcnt, [idx], ones, mask=neg)
        plsc.addupdate_scatter(hsum, [idx], 1.0 + pc, mask=neg)
        plsc.addupdate_scatter(accbuf, [lane], pc)

    for ci in range(NCHUNK):
        base = wid * PER_W + ci * CHUNK
        pltpu.sync_copy(pred_hbm.at[pl.ds(base, CHUNK)], pbuf)
        pltpu.sync_copy(tgt_hbm.at[pl.ds(base, CHUNK)], tbuf)
        plsc.parallel_loop(0, CHUNK // L, unroll=8)(_step)
    pltpu.sync_copy(hcnt, out_cnt.at[wid])
    pltpu.sync_copy(hsum, out_sum.at[wid])
    pltpu.sync_copy(accbuf, out_acc.at[wid])


def _combine_body(cnt_ref, sum_ref, acc_ref, out_ref):
    nf = float(N)
    kbf = float(KB)
    cnt = jnp.sum(cnt_ref[...], axis=0)
    ssum = jnp.sum(sum_ref[...], axis=0)
    acc_pc = jnp.sum(acc_ref[...])
    n_neg = jnp.sum(cnt)
    s_neg = jnp.sum(ssum)
    gts = nf - n_neg
    s_pos = (nf - acc_pc) - (2.0 * n_neg - s_neg)
    term1 = s_pos / (nf + EPS)

    iu0 = lax.broadcasted_iota(jnp.int32, (128, 128), 0)
    iu1 = lax.broadcasted_iota(jnp.int32, (128, 128), 1)
    upper = (iu0 <= iu1).astype(jnp.float32)
    im0 = lax.broadcasted_iota(jnp.int32, (16, 16), 0)
    im1 = lax.broadcasted_iota(jnp.int32, (16, 16), 1)
    strict_lower = (im0 > im1).astype(jnp.float32)
    ones128 = jnp.ones((128, 128), jnp.float32)
    incl = jnp.dot(cnt, upper, preferred_element_type=jnp.float32)
    rowtot_b = jnp.dot(cnt, ones128, preferred_element_type=jnp.float32)
    excl_rows = jnp.dot(strict_lower, rowtot_b,
                        preferred_element_type=jnp.float32)
    j_excl = excl_rows + incl - cnt

    a = gts + j_excl + EPS
    term2 = jnp.sum(gts * ssum / (a * (a + cnt)))

    bidx = (lax.broadcasted_iota(jnp.int32, (16, 128), 0) * 128
            + lax.broadcasted_iota(jnp.int32, (16, 128), 1)).astype(jnp.float32)
    emax = jnp.max(jnp.where(cnt > 0.0, 1.0 + (kbf - bidx) / kbf, -1.0))
    loss = term1 + term2 + jnp.where(gts == 0.0, emax, 0.0)
    out_ref[0, 0] = loss


_combine = pl.pallas_call(
    _combine_body,
    out_shape=jax.ShapeDtypeStruct((1, 1), jnp.float32),
    out_specs=pl.BlockSpec(memory_space=pltpu.SMEM),
)


def kernel(pred, target):
    predf = pred.reshape(N)
    tgt = target.reshape(N)
    cnt, ssum, acc = _build_sc_hist()(predf, tgt)
    cnt3 = cnt.reshape(NW * L, KB // 128, 128)
    sum3 = ssum.reshape(NW * L, KB // 128, 128)
    loss = _combine(cnt3, sum3, acc)
    return loss[0, 0]


# speedup vs baseline: 57.4113x; 1.3263x over previous
import functools

import jax
import jax.numpy as jnp
from jax import lax
from jax.experimental import pallas as pl
from jax.experimental.pallas import tpu as pltpu
from jax.experimental.pallas import tpu_sc as plsc

N = 8 * 512 * 512
NW = 32
ROWS_W = 512 // 4
RCHUNK = 32
NCHUNK = ROWS_W // RCHUNK
KB = 2048
L = 16
HIST = L * KB
EPS = 1e-10

@functools.cache
def _build_sc_hist():
    mesh = plsc.VectorSubcoreMesh(core_axis_name="c", subcore_axis_name="s")
    return functools.partial(
        pl.kernel,
        mesh=mesh,
        out_type=[
            jax.ShapeDtypeStruct((NW, HIST), jnp.float32),
            jax.ShapeDtypeStruct((NW, HIST), jnp.float32),
            jax.ShapeDtypeStruct((NW, L), jnp.float32),
        ],
        scratch_types=[
            pltpu.VMEM((RCHUNK, 512), jnp.float32),
            pltpu.VMEM((RCHUNK, 512), jnp.int32),
            pltpu.VMEM((HIST,), jnp.float32),
            pltpu.VMEM((HIST,), jnp.float32),
            pltpu.VMEM((L,), jnp.float32),
        ],
        compiler_params=pltpu.CompilerParams(needs_layout_passes=False),
    )(_sc_hist_body)


def _sc_hist_body(pred_hbm, tgt_hbm, out_cnt, out_sum, out_acc,
                  pbuf, tbuf, hcnt, hsum, accbuf):
    wid = lax.axis_index("c") * 16 + lax.axis_index("s")
    zeros = jnp.zeros((L,), jnp.float32)
    ones = jnp.ones((L,), jnp.float32)
    lane = lax.iota(jnp.int32, L)
    idxbase = lane * KB + (KB - 1)

    accbuf[...] = zeros

    @plsc.parallel_loop(0, HIST // L, unroll=8)
    def _zero(i):
        hcnt[pl.ds(i * L, L)] = zeros
        hsum[pl.ds(i * L, L)] = zeros

    def _step(i):
        r = i >> 5
        c = (i & 31) * L
        vp = pbuf[r, pl.ds(c, L)]
        vt = tbuf[r, pl.ds(c, L)]
        pc = jnp.minimum(jnp.maximum(vp, 0.0), 1.0)
        neg = vt == 0
        b = jnp.minimum((pc * float(KB)).astype(jnp.int32), KB - 1)
        idx = idxbase - b
        plsc.addupdate_scatter(hcnt, [idx], ones, mask=neg)
        plsc.addupdate_scatter(hsum, [idx], 1.0 + pc, mask=neg)
        plsc.addupdate_scatter(accbuf, [lane], pc)

    img = wid >> 2
    row0 = (wid & 3) * ROWS_W
    for ci in range(NCHUNK):
        rbase = row0 + ci * RCHUNK
        pltpu.sync_copy(pred_hbm.at[img, pl.ds(rbase, RCHUNK), :], pbuf)
        pltpu.sync_copy(tgt_hbm.at[img, pl.ds(rbase, RCHUNK), :], tbuf)
        plsc.parallel_loop(0, RCHUNK * 512 // L, unroll=8)(_step)
    pltpu.sync_copy(hcnt, out_cnt.at[wid])
    pltpu.sync_copy(hsum, out_sum.at[wid])
    pltpu.sync_copy(accbuf, out_acc.at[wid])


def _combine_body(cnt_ref, sum_ref, acc_ref, out_ref):
    nf = float(N)
    kbf = float(KB)
    cnt = jnp.sum(cnt_ref[...], axis=0)
    ssum = jnp.sum(sum_ref[...], axis=0)
    acc_pc = jnp.sum(acc_ref[...])
    n_neg = jnp.sum(cnt)
    s_neg = jnp.sum(ssum)
    gts = nf - n_neg
    s_pos = (nf - acc_pc) - (2.0 * n_neg - s_neg)
    term1 = s_pos / (nf + EPS)

    iu0 = lax.broadcasted_iota(jnp.int32, (128, 128), 0)
    iu1 = lax.broadcasted_iota(jnp.int32, (128, 128), 1)
    upper = (iu0 <= iu1).astype(jnp.float32)
    im0 = lax.broadcasted_iota(jnp.int32, (16, 16), 0)
    im1 = lax.broadcasted_iota(jnp.int32, (16, 16), 1)
    strict_lower = (im0 > im1).astype(jnp.float32)
    ones128 = jnp.ones((128, 128), jnp.float32)
    incl = jnp.dot(cnt, upper, preferred_element_type=jnp.float32)
    rowtot_b = jnp.dot(cnt, ones128, preferred_element_type=jnp.float32)
    excl_rows = jnp.dot(strict_lower, rowtot_b,
                        preferred_element_type=jnp.float32)
    j_excl = excl_rows + incl - cnt

    a = gts + j_excl + EPS
    term2 = jnp.sum(gts * ssum / (a * (a + cnt)))

    bidx = (lax.broadcasted_iota(jnp.int32, (16, 128), 0) * 128
            + lax.broadcasted_iota(jnp.int32, (16, 128), 1)).astype(jnp.float32)
    emax = jnp.max(jnp.where(cnt > 0.0, 1.0 + (kbf - bidx) / kbf, -1.0))
    loss = term1 + term2 + jnp.where(gts == 0.0, emax, 0.0)
    out_ref[0, 0] = loss


_combine = pl.pallas_call(
    _combine_body,
    out_shape=jax.ShapeDtypeStruct((1, 1), jnp.float32),
    out_specs=pl.BlockSpec(memory_space=pltpu.SMEM),
)


def kernel(pred, target):
    cnt, ssum, acc = _build_sc_hist()(pred, target)
    cnt3 = cnt.reshape(NW * L, KB // 128, 128)
    sum3 = ssum.reshape(NW * L, KB // 128, 128)
    loss = _combine(cnt3, sum3, acc)
    return loss[0, 0]


# speedup vs baseline: 70.5755x; 1.2293x over previous
import functools

import jax
import jax.numpy as jnp
from jax import lax
from jax.experimental import pallas as pl
from jax.experimental.pallas import tpu as pltpu
from jax.experimental.pallas import tpu_sc as plsc

N = 8 * 512 * 512
NW = 32
ROWS_W = 512 // 4
RCHUNK = 32
NCHUNK = ROWS_W // RCHUNK
KB = 2048
L = 16
HIST = L * KB
EPS = 1e-10


@functools.cache
def _build_sc_hist():
    mesh = plsc.VectorSubcoreMesh(core_axis_name="c", subcore_axis_name="s")
    return functools.partial(
        pl.kernel,
        mesh=mesh,
        out_type=jax.ShapeDtypeStruct((NW, HIST), jnp.float32),
        scratch_types=[
            pltpu.VMEM((RCHUNK, 512), jnp.float32),
            pltpu.VMEM((RCHUNK, 512), jnp.int32),
            pltpu.VMEM((HIST,), jnp.float32),
        ],
        compiler_params=pltpu.CompilerParams(needs_layout_passes=False),
    )(_sc_hist_body)


def _sc_hist_body(pred_hbm, tgt_hbm, out_cnt, pbuf, tbuf, hcnt):
    wid = lax.axis_index("c") * 16 + lax.axis_index("s")
    zeros = jnp.zeros((L,), jnp.float32)
    ones = jnp.ones((L,), jnp.float32)
    idxbase = lax.iota(jnp.int32, L) * KB + (KB - 1)

    @plsc.parallel_loop(0, HIST // L, unroll=8)
    def _zero(i):
        hcnt[pl.ds(i * L, L)] = zeros

    def _step(i):
        r = i >> 5
        c = (i & 31) * L
        vp = pbuf[r, pl.ds(c, L)]
        vt = tbuf[r, pl.ds(c, L)]
        b = jnp.minimum((vp * float(KB)).astype(jnp.int32), KB - 1)
        b = jnp.maximum(b, 0)
        idx = idxbase - b
        plsc.addupdate_scatter(hcnt, [idx], ones, mask=vt == 0)

    img = wid >> 2
    row0 = (wid & 3) * ROWS_W
    for ci in range(NCHUNK):
        rbase = row0 + ci * RCHUNK
        pltpu.sync_copy(pred_hbm.at[img, pl.ds(rbase, RCHUNK), :], pbuf)
        pltpu.sync_copy(tgt_hbm.at[img, pl.ds(rbase, RCHUNK), :], tbuf)
        plsc.parallel_loop(0, RCHUNK * 512 // L, unroll=8)(_step)
    pltpu.sync_copy(hcnt, out_cnt.at[wid])


def _sums_body(pred_ref, tgt_ref, out_ref):
    pc = jnp.clip(pred_ref[...], 0.0, 1.0)
    neg = tgt_ref[...] == 0
    out_ref[0, 0] = jnp.sum(pc)
    out_ref[0, 1] = jnp.sum(jnp.where(neg, pc, 0.0))


_sums = pl.pallas_call(
    _sums_body,
    out_shape=jax.ShapeDtypeStruct((1, 2), jnp.float32),
    out_specs=pl.BlockSpec(memory_space=pltpu.SMEM),
)


def _combine_body(cnt_ref, sums_ref, out_ref):
    nf = float(N)
    kbf = float(KB)
    cnt = jnp.sum(cnt_ref[...], axis=0)
    acc_pc = sums_ref[0, 0]
    neg_pc = sums_ref[0, 1]
    n_neg = jnp.sum(cnt)
    gts = nf - n_neg
    s_pos = gts - (acc_pc - neg_pc)
    term1 = s_pos / (nf + EPS)

    iu0 = lax.broadcasted_iota(jnp.int32, (128, 128), 0)
    iu1 = lax.broadcasted_iota(jnp.int32, (128, 128), 1)
    upper = (iu0 <= iu1).astype(jnp.float32)
    im0 = lax.broadcasted_iota(jnp.int32, (16, 16), 0)
    im1 = lax.broadcasted_iota(jnp.int32, (16, 16), 1)
    strict_lower = (im0 > im1).astype(jnp.float32)
    ones128 = jnp.ones((128, 128), jnp.float32)
    incl = jnp.dot(cnt, upper, preferred_element_type=jnp.float32)
    rowtot_b = jnp.dot(cnt, ones128, preferred_element_type=jnp.float32)
    excl_rows = jnp.dot(strict_lower, rowtot_b,
                        preferred_element_type=jnp.float32)
    j_excl = excl_rows + incl - cnt

    bidx = (lax.broadcasted_iota(jnp.int32, (16, 128), 0) * 128
            + lax.broadcasted_iota(jnp.int32, (16, 128), 1)).astype(jnp.float32)
    emid = 1.0 + (kbf - 0.5 - bidx) / kbf
    a = gts + j_excl + EPS
    term2 = jnp.sum(gts * (cnt * emid) / (a * (a + cnt)))

    emax = jnp.max(jnp.where(cnt > 0.0, 1.0 + (kbf - bidx) / kbf, -1.0))
    loss = term1 + term2 + jnp.where(gts == 0.0, emax, 0.0)
    out_ref[0, 0] = loss


_combine = pl.pallas_call(
    _combine_body,
    out_shape=jax.ShapeDtypeStruct((1, 1), jnp.float32),
    out_specs=pl.BlockSpec(memory_space=pltpu.SMEM),
)


def kernel(pred, target):
    cnt = _build_sc_hist()(pred, target)
    sums = _sums(pred, target)
    cnt3 = cnt.reshape(NW * L, KB // 128, 128)
    loss = _combine(cnt3, sums)
    return loss[0, 0]
